# Initial kernel scaffold; baseline (speedup 1.0000x reference)
import jax
import jax.numpy as jnp
from jax.experimental import pallas as pl


def kernel(query_scalar, doc_embeddings, W, top_k):
    raise NotImplementedError("write your pallas kernel here")


# speedup vs baseline: 1.9857x; 1.9857x over previous
import jax
import jax.numpy as jnp
import numpy as np
from jax.experimental import pallas as pl
from jax.experimental.pallas import tpu as pltpu

_N = 1_000_000
_D = 64
_K = 100
_B = 8192
_NB = (_N + _B - 1) // _B
_CH = 256
_CPB = _B // _CH
_NCH = _NB * _CPB

_NEG = np.float32(-np.inf)
_IMAX = np.int32(2147483647)


def _k1_scores(q_ref, doc_ref, scores_ref, cmax_ref):
    i = pl.program_id(0)
    sv = jnp.dot(doc_ref[...], q_ref[...],
                 preferred_element_type=jnp.float32)
    sc = sv.reshape(_CPB, _CH)
    row = jax.lax.broadcasted_iota(jnp.int32, (_CPB, _CH), 0)
    col = jax.lax.broadcasted_iota(jnp.int32, (_CPB, _CH), 1)
    gidx = i * _B + row * _CH + col
    sc = jnp.where(gidx < _N, sc, _NEG)
    scores_ref[...] = sc
    cmax_ref[...] = jnp.max(sc, axis=1).reshape(1, 1, _CPB)


def _k2_chunk_topk(cmax_ref, ids_ref):
    vals = cmax_ref[...].reshape(_NB, _CPB)
    cid = (jax.lax.broadcasted_iota(jnp.int32, (_NB, _CPB), 0) * _CPB
           + jax.lax.broadcasted_iota(jnp.int32, (_NB, _CPB), 1))

    def body(t, v):
        m = jnp.max(v)
        j = jnp.min(jnp.where(v == m, cid, _IMAX))
        ids_ref[t] = j
        return jnp.where(cid == j, _NEG, v)

    jax.lax.fori_loop(0, _K, body, vals)


def _k3_gather(ids_ref, sc_ref, cv_ref, ci_ref):
    i = pl.program_id(0)
    cv_ref[...] = sc_ref[...]
    lane = jax.lax.broadcasted_iota(jnp.int32, (1, 1, _CH), 2)
    ci_ref[...] = ids_ref[i] * _CH + lane


def _k4_final_topk(cv_ref, ci_ref, os_ref, oi_ref):
    vals = cv_ref[...].reshape(_K, _CH)
    idxs = ci_ref[...].reshape(_K, _CH)

    def body(t, v):
        m = jnp.max(v)
        j = jnp.min(jnp.where(v == m, idxs, _IMAX))
        os_ref[t] = m
        oi_ref[t] = j
        return jnp.where(idxs == j, _NEG, v)

    jax.lax.fori_loop(0, _K, body, vals)


def _run(query_scalar, doc_embeddings, W):
    vec = jnp.broadcast_to(query_scalar.reshape(()), (_D,))
    q = (vec @ W).reshape(_D, 1)
    scores, cmax = pl.pallas_call(
        _k1_scores,
        grid=(_NB,),
        in_specs=[
            pl.BlockSpec((_D, 1), lambda i: (0, 0)),
            pl.BlockSpec((_B, _D), lambda i: (i, 0)),
        ],
        out_specs=[
            pl.BlockSpec((_CPB, _CH), lambda i: (i, 0)),
            pl.BlockSpec((1, 1, _CPB), lambda i: (i, 0, 0)),
        ],
        out_shape=[
            jax.ShapeDtypeStruct((_NCH, _CH), jnp.float32),
            jax.ShapeDtypeStruct((_NB, 1, _CPB), jnp.float32),
        ],
        compiler_params=pltpu.CompilerParams(
            dimension_semantics=("arbitrary",)),
    )(q, doc_embeddings)

    ids = pl.pallas_call(
        _k2_chunk_topk,
        out_specs=pl.BlockSpec(memory_space=pltpu.SMEM),
        out_shape=jax.ShapeDtypeStruct((_K,), jnp.int32),
    )(cmax)

    cand_v, cand_i = pl.pallas_call(
        _k3_gather,
        grid_spec=pltpu.PrefetchScalarGridSpec(
            num_scalar_prefetch=1,
            grid=(_K,),
            in_specs=[pl.BlockSpec((1, 1, _CH), lambda i, ids: (ids[i], 0, 0))],
            out_specs=[
                pl.BlockSpec((1, 1, _CH), lambda i, ids: (i, 0, 0)),
                pl.BlockSpec((1, 1, _CH), lambda i, ids: (i, 0, 0)),
            ],
        ),
        out_shape=[
            jax.ShapeDtypeStruct((_K, 1, _CH), jnp.float32),
            jax.ShapeDtypeStruct((_K, 1, _CH), jnp.int32),
        ],
    )(ids, scores.reshape(_NCH, 1, _CH))

    top_s, top_i = pl.pallas_call(
        _k4_final_topk,
        out_specs=[
            pl.BlockSpec(memory_space=pltpu.SMEM),
            pl.BlockSpec(memory_space=pltpu.SMEM),
        ],
        out_shape=[
            jax.ShapeDtypeStruct((_K,), jnp.float32),
            jax.ShapeDtypeStruct((_K,), jnp.int32),
        ],
    )(cand_v, cand_i)
    return top_s, top_i


def kernel(query_scalar, doc_embeddings, W, top_k):
    top_s, top_i = _run(query_scalar, doc_embeddings, W)
    valid = jnp.arange(_K) < top_k
    return (jnp.where(valid, top_s, _NEG),
            jnp.where(valid, top_i, jnp.int32(-1)))


# speedup vs baseline: 2.1055x; 1.0603x over previous
import jax
import jax.numpy as jnp
import numpy as np
from jax.experimental import pallas as pl
from jax.experimental.pallas import tpu as pltpu

_N = 1_000_000
_D = 64
_K = 100
_CH = 200
_CPB = 40
_B = _CH * _CPB
_NB = _N // _B
_NCH = _NB * _CPB
_SR = 104

_NEG = np.float32(-np.inf)
_IMAX = np.int32(2147483647)


def _k1_cmax(q_ref, doc_ref, cmax_ref):
    sv = jnp.dot(doc_ref[...], q_ref[...],
                 preferred_element_type=jnp.float32)
    cmax_ref[...] = jnp.max(sv.reshape(1, _CPB, _CH, 1), axis=2)


def _k2_chunk_topk(cmax_ref, ids_ref):
    vals = cmax_ref[...].reshape(_NB, _CPB)
    cid = (jax.lax.broadcasted_iota(jnp.int32, (_NB, _CPB), 0) * _CPB
           + jax.lax.broadcasted_iota(jnp.int32, (_NB, _CPB), 1))

    def body(t, v):
        m = jnp.max(v)
        j = jnp.min(jnp.where(v == m, cid, _IMAX))
        ids_ref[t] = j
        return jnp.where(cid == j, _NEG, v)

    jax.lax.fori_loop(0, _K, body, vals)


def _k3_gather_select(ids_ref, q_ref, doc_ref, os_ref, oi_ref, sv_ref, si_ref):
    i = pl.program_id(0)

    @pl.when(i == 0)
    def _init():
        sv_ref[...] = jnp.full((_SR, _CH), _NEG, jnp.float32)

    sv = jnp.dot(doc_ref[...], q_ref[...],
                 preferred_element_type=jnp.float32)
    sv_ref[pl.ds(i, 1), :] = sv.reshape(1, _CH)
    si_ref[pl.ds(i, 1), :] = (
        ids_ref[i] * _CH
        + jax.lax.broadcasted_iota(jnp.int32, (1, _CH), 1))

    @pl.when(i == _K - 1)
    def _select():
        idxs = si_ref[...]

        def body(t, v):
            m = jnp.max(v)
            j = jnp.min(jnp.where(v == m, idxs, _IMAX))
            os_ref[t] = m
            oi_ref[t] = j
            return jnp.where(idxs == j, _NEG, v)

        jax.lax.fori_loop(0, _K, body, sv_ref[...])


def _run(query_scalar, doc_embeddings, W):
    vec = jnp.broadcast_to(query_scalar.reshape(()), (_D,))
    q = (vec @ W).reshape(_D, 1)

    cmax = pl.pallas_call(
        _k1_cmax,
        grid=(_NB,),
        in_specs=[
            pl.BlockSpec((_D, 1), lambda i: (0, 0)),
            pl.BlockSpec((_B, _D), lambda i: (i, 0)),
        ],
        out_specs=pl.BlockSpec((1, _CPB, 1), lambda i: (i, 0, 0)),
        out_shape=jax.ShapeDtypeStruct((_NB, _CPB, 1), jnp.float32),
        compiler_params=pltpu.CompilerParams(
            dimension_semantics=("arbitrary",)),
    )(q, doc_embeddings)

    ids = pl.pallas_call(
        _k2_chunk_topk,
        out_specs=pl.BlockSpec(memory_space=pltpu.SMEM),
        out_shape=jax.ShapeDtypeStruct((_K,), jnp.int32),
    )(cmax)

    top_s, top_i = pl.pallas_call(
        _k3_gather_select,
        grid_spec=pltpu.PrefetchScalarGridSpec(
            num_scalar_prefetch=1,
            grid=(_K,),
            in_specs=[
                pl.BlockSpec((_D, 1), lambda i, ids: (0, 0)),
                pl.BlockSpec((_CH, _D), lambda i, ids: (ids[i], 0)),
            ],
            out_specs=[
                pl.BlockSpec(memory_space=pltpu.SMEM),
                pl.BlockSpec(memory_space=pltpu.SMEM),
            ],
            scratch_shapes=[
                pltpu.VMEM((_SR, _CH), jnp.float32),
                pltpu.VMEM((_SR, _CH), jnp.int32),
            ],
        ),
        out_shape=[
            jax.ShapeDtypeStruct((_K,), jnp.float32),
            jax.ShapeDtypeStruct((_K,), jnp.int32),
        ],
    )(ids, q, doc_embeddings)
    return top_s, top_i


def kernel(query_scalar, doc_embeddings, W, top_k):
    top_s, top_i = _run(query_scalar, doc_embeddings, W)
    valid = jnp.arange(_K) < top_k
    return (jnp.where(valid, top_s, _NEG),
            jnp.where(valid, top_i, jnp.int32(-1)))


# speedup vs baseline: 2.2666x; 1.0765x over previous
import jax
import jax.numpy as jnp
import numpy as np
from jax.experimental import pallas as pl
from jax.experimental.pallas import tpu as pltpu

_N = 1_000_000
_D = 64
_K = 100
_CH = 200
_CPB = 100
_B = _CH * _CPB
_NB = _N // _B
_NCH = _NB * _CPB
_MR = 56
_SR = 104

_NEG = np.float32(-np.inf)
_IMAX = np.int32(2147483647)


def _ka_cmax_select(q_ref, doc_ref, ids_ref, cms_ref):
    i = pl.program_id(0)

    @pl.when(i == 0)
    def _init():
        cms_ref[...] = jnp.full((_MR, _CPB), _NEG, jnp.float32)

    sv = jnp.dot(doc_ref[...], q_ref[...],
                 preferred_element_type=jnp.float32)
    cm = jnp.max(sv.reshape(_CPB, _CH, 1), axis=1)
    cms_ref[pl.ds(i, 1), :] = cm.T

    @pl.when(i == _NB - 1)
    def _select():
        cid = (jax.lax.broadcasted_iota(jnp.int32, (_MR, _CPB), 0) * _CPB
               + jax.lax.broadcasted_iota(jnp.int32, (_MR, _CPB), 1))

        def body(t, v):
            m = jnp.max(v)
            j = jnp.min(jnp.where(v == m, cid, _IMAX))
            ids_ref[t] = j
            return jnp.where(cid == j, _NEG, v)

        jax.lax.fori_loop(0, _K, body, cms_ref[...])


def _kb_gather_select(ids_ref, tk_ref, q_ref, doc_ref, os_ref, oi_ref,
                      sv_ref, si_ref):
    i = pl.program_id(0)

    @pl.when(i == 0)
    def _init():
        sv_ref[...] = jnp.full((_SR, _CH), _NEG, jnp.float32)

    sv = jnp.dot(doc_ref[...], q_ref[...],
                 preferred_element_type=jnp.float32)
    sv_ref[pl.ds(i, 1), :] = sv.T
    si_ref[pl.ds(i, 1), :] = (
        ids_ref[i] * _CH
        + jax.lax.broadcasted_iota(jnp.int32, (1, _CH), 1))

    @pl.when(i == _K - 1)
    def _select():
        idxs = si_ref[...]
        tk = tk_ref[0]

        def body(t, v):
            m = jnp.max(v)
            j = jnp.min(jnp.where(v == m, idxs, _IMAX))
            valid = t < tk
            os_ref[t] = jnp.where(valid, m, _NEG)
            oi_ref[t] = jnp.where(valid, j, jnp.int32(-1))
            return jnp.where(idxs == j, _NEG, v)

        jax.lax.fori_loop(0, _K, body, sv_ref[...])


def kernel(query_scalar, doc_embeddings, W, top_k):
    vec = jnp.broadcast_to(query_scalar.reshape(()), (_D,))
    q = (vec @ W).reshape(_D, 1)
    tk = jnp.full((1,), top_k, jnp.int32)

    ids = pl.pallas_call(
        _ka_cmax_select,
        grid=(_NB,),
        in_specs=[
            pl.BlockSpec((_D, 1), lambda i: (0, 0)),
            pl.BlockSpec((_B, _D), lambda i: (i, 0)),
        ],
        out_specs=pl.BlockSpec(memory_space=pltpu.SMEM),
        out_shape=jax.ShapeDtypeStruct((_K,), jnp.int32),
        scratch_shapes=[pltpu.VMEM((_MR, _CPB), jnp.float32)],
        compiler_params=pltpu.CompilerParams(
            dimension_semantics=("arbitrary",)),
    )(q, doc_embeddings)

    top_s, top_i = pl.pallas_call(
        _kb_gather_select,
        grid_spec=pltpu.PrefetchScalarGridSpec(
            num_scalar_prefetch=2,
            grid=(_K,),
            in_specs=[
                pl.BlockSpec((_D, 1), lambda i, ids, tk: (0, 0)),
                pl.BlockSpec((_CH, _D), lambda i, ids, tk: (ids[i], 0)),
            ],
            out_specs=[
                pl.BlockSpec(memory_space=pltpu.SMEM),
                pl.BlockSpec(memory_space=pltpu.SMEM),
            ],
            scratch_shapes=[
                pltpu.VMEM((_SR, _CH), jnp.float32),
                pltpu.VMEM((_SR, _CH), jnp.int32),
            ],
        ),
        out_shape=[
            jax.ShapeDtypeStruct((_K,), jnp.float32),
            jax.ShapeDtypeStruct((_K,), jnp.int32),
        ],
    )(ids, tk, q, doc_embeddings)
    return top_s, top_i


# speedup vs baseline: 2.4130x; 1.0646x over previous
import jax
import jax.numpy as jnp
import numpy as np
from jax.experimental import pallas as pl
from jax.experimental.pallas import tpu as pltpu

_N = 1_000_000
_D = 64
_K = 100
_CH = 200
_CPB = 100
_B = _CH * _CPB
_NB = _N // _B
_NCH = _NB * _CPB
_MR = 56
_SR = 104
_G = 4

_NEG = np.float32(-np.inf)
_IMAX = np.int32(2147483647)


def _ka_cmax_select(q_ref, doc_ref, ids_ref, cms_ref):
    i = pl.program_id(0)

    @pl.when(i == 0)
    def _init():
        cms_ref[...] = jnp.full((_MR, _CPB), _NEG, jnp.float32)

    sv = jnp.dot(doc_ref[...], q_ref[...],
                 preferred_element_type=jnp.float32)
    cm = jnp.max(sv.reshape(_CPB, _CH, 1), axis=1)
    cms_ref[pl.ds(i, 1), :] = cm.T

    @pl.when(i == _NB - 1)
    def _select():
        cid = (jax.lax.broadcasted_iota(jnp.int32, (_MR, _CPB), 0) * _CPB
               + jax.lax.broadcasted_iota(jnp.int32, (_MR, _CPB), 1))

        def body(t, v):
            m = jnp.max(v)
            j = jnp.min(jnp.where(v == m, cid, _IMAX))
            ids_ref[t] = j
            return jnp.where(cid == j, _NEG, v)

        jax.lax.fori_loop(0, _K, body, cms_ref[...])


def _kb_gather_select(ids_ref, tk_ref, q_ref, d0_ref, d1_ref, d2_ref, d3_ref,
                      os_ref, oi_ref, sv_ref, si_ref):
    i = pl.program_id(0)

    @pl.when(i == 0)
    def _init():
        sv_ref[...] = jnp.full((_SR, _CH), _NEG, jnp.float32)

    lane = jax.lax.broadcasted_iota(jnp.int32, (1, _CH), 1)
    for g, d_ref in enumerate((d0_ref, d1_ref, d2_ref, d3_ref)):
        sv = jnp.dot(d_ref[...], q_ref[...],
                     preferred_element_type=jnp.float32)
        r = i * _G + g
        sv_ref[pl.ds(r, 1), :] = sv.T
        si_ref[pl.ds(r, 1), :] = ids_ref[r] * _CH + lane

    @pl.when(i == _K // _G - 1)
    def _select():
        idxs = si_ref[...]
        tk = tk_ref[0]

        def body(t, v):
            m = jnp.max(v)
            j = jnp.min(jnp.where(v == m, idxs, _IMAX))
            valid = t < tk
            os_ref[t] = jnp.where(valid, m, _NEG)
            oi_ref[t] = jnp.where(valid, j, jnp.int32(-1))
            return jnp.where(idxs == j, _NEG, v)

        jax.lax.fori_loop(0, _K, body, sv_ref[...])


def kernel(query_scalar, doc_embeddings, W, top_k):
    vec = jnp.broadcast_to(query_scalar.reshape(()), (_D,))
    q = (vec @ W).reshape(_D, 1)
    tk = jnp.full((1,), top_k, jnp.int32)

    ids = pl.pallas_call(
        _ka_cmax_select,
        grid=(_NB,),
        in_specs=[
            pl.BlockSpec((_D, 1), lambda i: (0, 0)),
            pl.BlockSpec((_B, _D), lambda i: (i, 0)),
        ],
        out_specs=pl.BlockSpec(memory_space=pltpu.SMEM),
        out_shape=jax.ShapeDtypeStruct((_K,), jnp.int32),
        scratch_shapes=[pltpu.VMEM((_MR, _CPB), jnp.float32)],
        compiler_params=pltpu.CompilerParams(
            dimension_semantics=("arbitrary",)),
    )(q, doc_embeddings)

    top_s, top_i = pl.pallas_call(
        _kb_gather_select,
        grid_spec=pltpu.PrefetchScalarGridSpec(
            num_scalar_prefetch=2,
            grid=(_K // _G,),
            in_specs=[
                pl.BlockSpec((_D, 1), lambda i, ids, tk: (0, 0)),
                pl.BlockSpec((_CH, _D), lambda i, ids, tk: (ids[i * _G], 0)),
                pl.BlockSpec((_CH, _D),
                             lambda i, ids, tk: (ids[i * _G + 1], 0)),
                pl.BlockSpec((_CH, _D),
                             lambda i, ids, tk: (ids[i * _G + 2], 0)),
                pl.BlockSpec((_CH, _D),
                             lambda i, ids, tk: (ids[i * _G + 3], 0)),
            ],
            out_specs=[
                pl.BlockSpec(memory_space=pltpu.SMEM),
                pl.BlockSpec(memory_space=pltpu.SMEM),
            ],
            scratch_shapes=[
                pltpu.VMEM((_SR, _CH), jnp.float32),
                pltpu.VMEM((_SR, _CH), jnp.int32),
            ],
        ),
        out_shape=[
            jax.ShapeDtypeStruct((_K,), jnp.float32),
            jax.ShapeDtypeStruct((_K,), jnp.int32),
        ],
    )(ids, tk, q, doc_embeddings, doc_embeddings, doc_embeddings,
      doc_embeddings)
    return top_s, top_i


# speedup vs baseline: 2.4192x; 1.0026x over previous
import jax
import jax.numpy as jnp
import numpy as np
from jax.experimental import pallas as pl
from jax.experimental.pallas import tpu as pltpu

_N = 1_000_000
_D = 64
_K = 100
_CH = 200
_CPB = 100
_B = _CH * _CPB
_NB = _N // _B
_NCH = _NB * _CPB
_MR = 56
_SR = 104
_G = 4

_NEG = np.float32(-np.inf)
_IMAX = np.int32(2147483647)


def _ka_cmax_select(s_ref, w_ref, doc_ref, ids_ref, q_ref, qs_ref, cms_ref):
    i = pl.program_id(0)

    @pl.when(i == 0)
    def _init():
        cms_ref[...] = jnp.full((_MR, _CPB), _NEG, jnp.float32)
        vec = jnp.full((1, _D), s_ref[0], jnp.float32)
        qrow = jnp.dot(vec, w_ref[...],
                       preferred_element_type=jnp.float32)
        qs_ref[...] = qrow.T
        q_ref[...] = qrow.T

    sv = jnp.dot(doc_ref[...], qs_ref[...],
                 preferred_element_type=jnp.float32)
    cm = jnp.max(sv.reshape(_CPB, _CH, 1), axis=1)
    cms_ref[pl.ds(i, 1), :] = cm.T

    @pl.when(i == _NB - 1)
    def _select():
        cid = (jax.lax.broadcasted_iota(jnp.int32, (_MR, _CPB), 0) * _CPB
               + jax.lax.broadcasted_iota(jnp.int32, (_MR, _CPB), 1))

        def body(t, v):
            m = jnp.max(v)
            j = jnp.min(jnp.where(v == m, cid, _IMAX))
            ids_ref[t] = j
            return jnp.where(cid == j, _NEG, v)

        jax.lax.fori_loop(0, _K, body, cms_ref[...])


def _kb_gather_select(ids_ref, tk_ref, q_ref, d0_ref, d1_ref, d2_ref, d3_ref,
                      os_ref, oi_ref, sv_ref, si_ref):
    i = pl.program_id(0)

    @pl.when(i == 0)
    def _init():
        sv_ref[...] = jnp.full((_SR, _CH), _NEG, jnp.float32)

    lane = jax.lax.broadcasted_iota(jnp.int32, (1, _CH), 1)
    for g, d_ref in enumerate((d0_ref, d1_ref, d2_ref, d3_ref)):
        sv = jnp.dot(d_ref[...], q_ref[...],
                     preferred_element_type=jnp.float32)
        r = i * _G + g
        sv_ref[pl.ds(r, 1), :] = sv.T
        si_ref[pl.ds(r, 1), :] = ids_ref[r] * _CH + lane

    @pl.when(i == _K // _G - 1)
    def _select():
        idxs = si_ref[...]
        tk = tk_ref[0]

        def body(t, v):
            m = jnp.max(v)
            j = jnp.min(jnp.where(v == m, idxs, _IMAX))
            valid = t < tk
            os_ref[t] = jnp.where(valid, m, _NEG)
            oi_ref[t] = jnp.where(valid, j, jnp.int32(-1))
            return jnp.where(idxs == j, _NEG, v)

        jax.lax.fori_loop(0, _K, body, sv_ref[...])


def kernel(query_scalar, doc_embeddings, W, top_k):
    tk = jnp.full((1,), top_k, jnp.int32)

    ids, q = pl.pallas_call(
        _ka_cmax_select,
        grid=(_NB,),
        in_specs=[
            pl.BlockSpec(memory_space=pltpu.SMEM),
            pl.BlockSpec((_D, _D), lambda i: (0, 0)),
            pl.BlockSpec((_B, _D), lambda i: (i, 0)),
        ],
        out_specs=[
            pl.BlockSpec(memory_space=pltpu.SMEM),
            pl.BlockSpec((_D, 1), lambda i: (0, 0)),
        ],
        out_shape=[
            jax.ShapeDtypeStruct((_K,), jnp.int32),
            jax.ShapeDtypeStruct((_D, 1), jnp.float32),
        ],
        scratch_shapes=[
            pltpu.VMEM((_D, 1), jnp.float32),
            pltpu.VMEM((_MR, _CPB), jnp.float32),
        ],
        compiler_params=pltpu.CompilerParams(
            dimension_semantics=("arbitrary",)),
    )(query_scalar, W, doc_embeddings)

    top_s, top_i = pl.pallas_call(
        _kb_gather_select,
        grid_spec=pltpu.PrefetchScalarGridSpec(
            num_scalar_prefetch=2,
            grid=(_K // _G,),
            in_specs=[
                pl.BlockSpec((_D, 1), lambda i, ids, tk: (0, 0)),
                pl.BlockSpec((_CH, _D), lambda i, ids, tk: (ids[i * _G], 0)),
                pl.BlockSpec((_CH, _D),
                             lambda i, ids, tk: (ids[i * _G + 1], 0)),
                pl.BlockSpec((_CH, _D),
                             lambda i, ids, tk: (ids[i * _G + 2], 0)),
                pl.BlockSpec((_CH, _D),
                             lambda i, ids, tk: (ids[i * _G + 3], 0)),
            ],
            out_specs=[
                pl.BlockSpec(memory_space=pltpu.SMEM),
                pl.BlockSpec(memory_space=pltpu.SMEM),
            ],
            scratch_shapes=[
                pltpu.VMEM((_SR, _CH), jnp.float32),
                pltpu.VMEM((_SR, _CH), jnp.int32),
            ],
        ),
        out_shape=[
            jax.ShapeDtypeStruct((_K,), jnp.float32),
            jax.ShapeDtypeStruct((_K,), jnp.int32),
        ],
    )(ids, tk, q, doc_embeddings, doc_embeddings, doc_embeddings,
      doc_embeddings)
    return top_s, top_i
